# Initial kernel scaffold; baseline (speedup 1.0000x reference)
import jax
import jax.numpy as jnp
from jax.experimental import pallas as pl


def kernel(params, ab_x, ab_edge_index, ab_edge_attr, ba_edge_index, ba_edge_attr, node_id_atom, node_id_bond, edge_id_atom, edge_id_bond, atom_coords):
    raise NotImplementedError("write your pallas kernel here")


# speedup vs baseline: 1.0156x; 1.0156x over previous
import functools

import jax
import jax.numpy as jnp
from jax.experimental import pallas as pl

_N = 50000
_E = 800000
_A = 800000
_D = 32
_L = 8
_B = 128
_NA = 7
_NB = 3

_BLK = 2000


def _rbf(v, centers):
    return jnp.exp(-10.0 * (v[:, None] - centers[None, :]) ** 2)


def _mlp2(x, W1, b1, W2, b2):
    h = jax.nn.relu(x @ W1 + b1)
    return jax.nn.relu(h @ W2 + b2)


def _block_body(x_ref, aggr_ref, fac_ref, w1_ref, b1_ref, w2_ref, b2_ref,
                g_ref, be_ref, o_ref, *, last_act):
    x = x_ref[...]
    h = x + aggr_ref[...]
    h1 = jnp.maximum(jnp.dot(h, w1_ref[...],
                             preferred_element_type=jnp.float32) + b1_ref[...], 0.0)
    out = jnp.dot(h1, w2_ref[...],
                  preferred_element_type=jnp.float32) + b2_ref[...]
    mu = jnp.mean(out, axis=-1, keepdims=True)
    var = jnp.mean((out - mu) ** 2, axis=-1, keepdims=True)
    out = g_ref[...] * (out - mu) * jax.lax.rsqrt(var + 1e-5) + be_ref[...]
    out = out * fac_ref[...]
    if last_act:
        out = jnp.maximum(out, 0.0)
    o_ref[...] = out + x


@functools.partial(jax.jit, static_argnames=("last_act",))
def _block_tc(x, aggr, fac, W1, b1, W2, b2, g, be, last_act):
    rows = x.shape[0]
    assert rows % _BLK == 0
    grid = rows // _BLK
    row_spec = pl.BlockSpec((_BLK, _D), lambda i: (i, 0))
    fac_spec = pl.BlockSpec((_BLK, 1), lambda i: (i, 0))
    full = lambda s: pl.BlockSpec(s, lambda i: (0,) * len(s))
    return pl.pallas_call(
        functools.partial(_block_body, last_act=last_act),
        grid=(grid,),
        in_specs=[row_spec, row_spec, fac_spec,
                  full((_D, 2 * _D)), full((1, 2 * _D)),
                  full((2 * _D, _D)), full((1, _D)),
                  full((1, _D)), full((1, _D))],
        out_specs=row_spec,
        out_shape=jax.ShapeDtypeStruct((rows, _D), jnp.float32),
    )(x, aggr, fac, W1, b1.reshape(1, -1), W2, b2.reshape(1, -1),
      g.reshape(1, -1), be.reshape(1, -1))


def _segsum(vals, seg, num):
    return jax.ops.segment_sum(vals, seg, num_segments=num)


def kernel(params, ab_x, ab_edge_index, ab_edge_attr, ba_edge_index,
           ba_edge_attr, node_id_atom, node_id_bond, edge_id_atom,
           edge_id_bond, atom_coords):
    p = params
    len_centers = jnp.arange(0.0, 2.0, 0.1, dtype=jnp.float32)
    ang_centers = jnp.arange(0.0, jnp.pi, 0.1, dtype=jnp.float32)

    cat = ab_edge_attr[:, :_NB].astype(jnp.int32)
    blen = ab_edge_attr[:, _NB]
    angle = ba_edge_attr[:, 0]

    cnt_a = _segsum(jnp.ones((_N,), jnp.float32), node_id_atom, _B)
    fac_a = (1.0 / jnp.sqrt(jnp.maximum(cnt_a, 1.0)))[node_id_atom][:, None]
    cnt_b = _segsum(jnp.ones((_E,), jnp.float32), node_id_bond, _B)
    fac_b = (1.0 / jnp.sqrt(jnp.maximum(cnt_b, 1.0)))[node_id_bond][:, None]

    node_hidden = jnp.zeros((_N, _D), jnp.float32)
    for f in range(_NA):
        node_hidden = node_hidden + p["atom_emb"][f][ab_x[:, f]]

    bond_embed = jnp.zeros((_E, _D), jnp.float32)
    for f in range(_NB):
        bond_embed = bond_embed + p["init_bond_emb"][f][cat[:, f]]
    rbf_len = _rbf(blen, len_centers)
    rbf_ang = _rbf(angle, ang_centers)
    edge_hidden = bond_embed + rbf_len @ p["init_rbf_W"] + p["init_rbf_b"]

    pos_mlp = _mlp2(atom_coords, p["pos_W1"], p["pos_b1"], p["pos_W2"], p["pos_b2"])
    sent = atom_coords[ab_edge_index[0]]
    recv = atom_coords[ab_edge_index[1]]
    length = jnp.linalg.norm(sent - recv, axis=-1)[:, None]
    dis_mlp = _mlp2(length, p["dis_W1"], p["dis_b1"], p["dis_W2"], p["dis_b2"])

    ab_src, ab_dst = ab_edge_index[0], ab_edge_index[1]
    ba_src, ba_dst = ba_edge_index[0], ba_edge_index[1]

    nh = node_hidden
    eh = edge_hidden
    for l in range(_L):
        last_act = l != _L - 1
        x_n = nh + pos_mlp
        ea = eh + dis_mlp
        msg = jax.nn.relu(x_n[ab_src] + ea)
        aggr = _segsum(msg, ab_dst, _N)
        nh = _block_tc(x_n, aggr, fac_a, p["ab_W1"][l], p["ab_b1"][l],
                       p["ab_W2"][l], p["ab_b2"][l], p["ab_ln_g"][l],
                       p["ab_ln_b"][l], last_act)

        cur_edge = jnp.zeros((_E, _D), jnp.float32)
        for f in range(_NB):
            cur_edge = cur_edge + p["bond_emb"][l][f][cat[:, f]]
        cur_edge = cur_edge + rbf_len @ p["rbf_W"][l] + p["rbf_b"][l]
        cur_angle = rbf_ang @ p["ang_rbf_W"][l] + p["ang_rbf_b"][l]
        msg2 = jax.nn.relu(cur_edge[ba_src] + cur_angle)
        aggr2 = _segsum(msg2, ba_dst, _E)
        eh = _block_tc(cur_edge, aggr2, fac_b, p["ba_W1"][l], p["ba_b1"][l],
                       p["ba_W2"][l], p["ba_b2"][l], p["ba_ln_g"][l],
                       p["ba_ln_b"][l], last_act)
    return (nh, eh)
